# baseline (speedup 1.0000x reference)
import functools

import jax
import jax.numpy as jnp
from jax import lax
from jax.experimental import pallas as pl
from jax.experimental.pallas import tpu as pltpu
from jax.experimental.pallas import tpu_sc as plsc

_LEVELS = (128, 256, 512, 1024)
_NL = len(_LEVELS)
_C = 16
_N = 524288
_NC = 2
_NS = 16
_NW = _NC * _NS
_B = 128
_NP = _N // _NW
_NCHUNK = _NP // _B
_LANES = 16
_K = 512

_HW = tuple(w * w for w in _LEVELS)
_OFF = (0,) + tuple(sum(_HW[: i + 1]) for i in range(_NL))
_HWTOT = _OFF[-1]


def _sc_sample(x, y, g0, g1, g2, g3):
    mesh = plsc.VectorSubcoreMesh(core_axis_name="c", subcore_axis_name="s")

    vmem_i = lambda: pltpu.VMEM((_B,), jnp.int32)
    vmem_f = lambda: pltpu.VMEM((_B,), jnp.float32)

    def scratch_set():
        return [
            vmem_f(), vmem_f(),
            [[vmem_i() for _ in range(4)] for _ in range(_NL)],
            pltpu.VMEM((_B, _LANES), jnp.float32),
            [[pltpu.VMEM((_B, _C), jnp.float32) for _ in range(4)]
             for _ in range(_NL)],
            pltpu.VMEM((_B, _NL * _C), jnp.float32),
            pltpu.SemaphoreType.DMA,
        ]

    def relayout_set():
        return [
            pltpu.VMEM((_C, _K), jnp.float32),
            pltpu.VMEM((_K, _C), jnp.float32),
            pltpu.SemaphoreType.DMA,
        ]

    cp = pltpu.CompilerParams(
        needs_layout_passes=False, use_tc_tiling_on_sc=False)

    @functools.partial(
        pl.kernel,
        out_type=(
            jax.ShapeDtypeStruct((_N, _NL * _C), jnp.float32),
            jax.ShapeDtypeStruct((_NC * _HWTOT, _C), jnp.float32),
        ),
        mesh=mesh,
        compiler_params=cp,
        scratch_types=[scratch_set(), scratch_set(),
                       relayout_set(), relayout_set()],
    )
    def grid_sample_kernel(x_hbm, y_hbm, g0_hbm, g1_hbm, g2_hbm, g3_hbm,
                           out_hbm, tbl_hbm, set0, set1, rset0, rset1):
        g_hbm = (g0_hbm, g1_hbm, g2_hbm, g3_hbm)
        sets = (set0, set1)
        rsets = (rset0, rset1)
        cid = lax.axis_index("c")
        sid = lax.axis_index("s")
        wid = cid * _NS + sid
        base = wid * _NP
        tbase = cid * _HWTOT
        iota = lax.iota(jnp.int32, _LANES)

        for L in range(_NL):
            per_tile = _HW[L] // _NS
            pix0 = sid * per_tile
            nck = per_tile // _K

            def afire(c, s, L=L, pix0=pix0):
                ib, ob, sema = rsets[s]
                p0 = pix0 + c * _K
                for ch in range(_C):
                    pltpu.async_copy(
                        g_hbm[L].at[ch, pl.ds(p0, _K)], ib.at[ch], sema)

            def adrain(c, s, L=L, pix0=pix0):
                ib, ob, sema = rsets[s]
                p0 = pix0 + c * _K
                for ch in range(_C):
                    pltpu.make_async_copy(
                        g_hbm[L].at[ch, pl.ds(p0, _K)], ib.at[ch],
                        sema).wait()

                @plsc.parallel_loop(0, _K, step=1, unroll=2)
                def _t(p):
                    v = plsc.load_gather(ib, [iota, iota * 0 + p])
                    ob[p, pl.ds(0, _C)] = v

                pltpu.sync_copy(
                    ob, tbl_hbm.at[pl.ds(tbase + _OFF[L] + p0, _K)])

            afire(0, 0)
            afire(1, 1)

            @pl.loop(0, nck // 2 - 1)
            def _asteady(i):
                c0 = 2 * i
                adrain(c0, 0)
                afire(c0 + 2, 0)
                adrain(c0 + 1, 1)
                afire(c0 + 3, 1)

            adrain(nck - 2, 0)
            adrain(nck - 1, 1)

        plsc.subcore_barrier()

        def fire(c, s):
            xv, yv, idx, wts, rows, out_v, semg = sets[s]
            coff = base + c * _B
            pltpu.sync_copy(x_hbm.at[pl.ds(coff, _B)], xv)
            pltpu.sync_copy(y_hbm.at[pl.ds(coff, _B)], yv)

            for L in range(_NL):
                w = _LEVELS[L]
                hw = (w - 1) * 0.5
                gbase = tbase + _OFF[L]

                @plsc.parallel_loop(0, _B, step=_LANES, unroll=1)
                def _ixw(i):
                    sl = pl.ds(i, _LANES)
                    ridx = iota + i
                    sx = jnp.clip(xv[sl] * hw + hw, 0.0, w - 1.0)
                    sy = jnp.clip(yv[sl] * hw + hw, 0.0, w - 1.0)
                    x0 = sx.astype(jnp.int32)
                    y0 = sy.astype(jnp.int32)
                    fx = sx - x0.astype(jnp.float32)
                    fy = sy - y0.astype(jnp.float32)
                    dx = jnp.minimum(x0 + 1, w - 1) - x0
                    dy = (jnp.minimum(y0 + 1, w - 1) - y0) * w
                    b00 = y0 * w + x0 + gbase
                    idx[L][0][sl] = b00
                    idx[L][1][sl] = b00 + dx
                    idx[L][2][sl] = b00 + dy
                    idx[L][3][sl] = b00 + dy + dx
                    gx = 1.0 - fx
                    gy = 1.0 - fy
                    for k, wk in enumerate((gx * gy, fx * gy, gx * fy,
                                            fx * fy)):
                        col = jnp.full((_LANES,), 4 * L + k, jnp.int32)
                        plsc.store_scatter(wts, [ridx, col], wk)

            for L in range(_NL):
                for cnr in range(4):
                    pltpu.async_copy(
                        tbl_hbm.at[idx[L][cnr]], rows[L][cnr], semg)

        def blend(c, s):
            xv, yv, idx, wts, rows, out_v, semg = sets[s]
            for L in range(_NL):
                for cnr in range(4):
                    pltpu.make_async_copy(
                        tbl_hbm.at[idx[L][cnr]], rows[L][cnr], semg).wait()

            @plsc.parallel_loop(0, _B, step=1, unroll=2)
            def _blend(i):
                wv = wts[i]
                for L in range(_NL):
                    acc = (rows[L][0][i] * wv[4 * L]
                           + rows[L][1][i] * wv[4 * L + 1]
                           + rows[L][2][i] * wv[4 * L + 2]
                           + rows[L][3][i] * wv[4 * L + 3])
                    out_v[i, pl.ds(L * _C, _C)] = acc

            coff = base + c * _B
            pltpu.sync_copy(out_v, out_hbm.at[pl.ds(coff, _B)])

        fire(0, 0)
        fire(1, 1)

        @pl.loop(0, _NCHUNK // 2 - 1)
        def _steady(i):
            c0 = 2 * i
            blend(c0, 0)
            fire(c0 + 2, 0)
            blend(c0 + 1, 1)
            fire(c0 + 3, 1)

        blend(_NCHUNK - 2, 0)
        blend(_NCHUNK - 1, 1)

    return grid_sample_kernel(x, y, g0, g1, g2, g3)


def kernel(xy, grid_0, grid_1, grid_2, grid_3):
    x = xy[:, 0] + 0.0
    y = xy[:, 1] + 0.0
    grids = [
        g.reshape(_C, -1) for g in (grid_0, grid_1, grid_2, grid_3)
    ]
    out, _ = _sc_sample(x, y, *grids)
    return out
